# Initial kernel scaffold; baseline (speedup 1.0000x reference)
import jax
import jax.numpy as jnp
from jax.experimental import pallas as pl


def kernel(pred_next_dt, next_dt, dt, offsets):
    raise NotImplementedError("write your pallas kernel here")


# speedup vs baseline: 1.1006x; 1.1006x over previous
import functools

import jax
import jax.numpy as jnp
from jax import lax
from jax.experimental import pallas as pl
from jax.experimental.pallas import tpu as pltpu
from jax.experimental.pallas import tpu_sc as plsc

_L = 16
_NS = 16
_EPS = 1e-5
_LN2 = 0.6931471805599453
_SQRT2 = 1.4142135623730951


def _vlog(x):
    bits = lax.bitcast_convert_type(x, jnp.int32)
    e = (bits >> 23) - 127
    m = lax.bitcast_convert_type((bits & 0x007FFFFF) | 0x3F800000, jnp.float32)
    big = m > _SQRT2
    e = jnp.where(big, e + 1, e)
    m = jnp.where(big, m * 0.5, m)
    t = (m - 1.0) / (m + 1.0)
    t2 = t * t
    p = 2.0 * t * (1.0 + t2 * (1.0 / 3.0 + t2 * (0.2 + t2 * (1.0 / 7.0))))
    return e.astype(jnp.float32) * _LN2 + p


def _term(tau, dtv):
    return -_vlog(tau) - (dtv + _EPS) / tau


def _make_sc_call(n, nseg, denom):
    chunk = n // _NS
    nvec = chunk // _L
    mesh = plsc.VectorSubcoreMesh(
        core_axis_name="c", subcore_axis_name="s", num_cores=2,
        num_subcores=_NS)

    @functools.partial(
        pl.kernel,
        out_type=jax.ShapeDtypeStruct((_L,), jnp.float32),
        mesh=mesh,
        compiler_params=pltpu.CompilerParams(needs_layout_passes=False),
        scratch_types=dict(
            tau_v=pltpu.VMEM((chunk,), jnp.float32),
            dt_v=pltpu.VMEM((chunk,), jnp.float32),
            s_v=pltpu.VMEM((_L,), jnp.int32),
            e_v=pltpu.VMEM((_L,), jnp.int32),
            ilo_v=pltpu.VMEM((_L,), jnp.int32),
            ihi_v=pltpu.VMEM((_L,), jnp.int32),
            g_tlo=pltpu.VMEM((_L,), jnp.float32),
            g_thi=pltpu.VMEM((_L,), jnp.float32),
            g_dlo=pltpu.VMEM((_L,), jnp.float32),
            g_dhi=pltpu.VMEM((_L,), jnp.float32),
            acc_v=pltpu.VMEM((_L,), jnp.float32),
            red_v=pltpu.VMEM((_NS * _L,), jnp.float32),
            out_v=pltpu.VMEM((_L,), jnp.float32),
            shared=pltpu.VMEM_SHARED((_NS * _L,), jnp.float32),
            sem=pltpu.SemaphoreType.DMA,
        ),
    )
    def sc_loss(tau_hbm, dt_hbm, s_hbm, e_hbm, out_hbm, *, tau_v, dt_v, s_v,
                e_v, ilo_v, ihi_v, g_tlo, g_thi, g_dlo, g_dhi, acc_v, red_v,
                out_v, shared, sem):
        cid = lax.axis_index("c")
        sid = lax.axis_index("s")

        base = sid * chunk
        pltpu.sync_copy(tau_hbm.at[pl.ds(base, chunk)], tau_v)
        pltpu.sync_copy(dt_hbm.at[pl.ds(base, chunk)], dt_v)
        acc = jnp.zeros((_L,), jnp.float32)
        for k in range(nvec):
            tv = tau_v[pl.ds(k * _L, _L)]
            dv = dt_v[pl.ds(k * _L, _L)]
            acc = acc + _term(tv, dv)
        acc_v[...] = acc
        pltpu.sync_copy(acc_v, shared.at[pl.ds(sid * _L, _L)])
        plsc.subcore_barrier()

        @pl.when(jnp.logical_and(cid == 0, sid == 0))
        def _finalize():
            pltpu.sync_copy(s_hbm, s_v)
            pltpu.sync_copy(e_hbm, e_v)
            sv = s_v[...]
            ev = e_v[...]
            ilo_v[...] = sv
            ihi_v[...] = jnp.maximum(ev - 1, 0)
            pltpu.async_copy(tau_hbm.at[ilo_v], g_tlo, sem).wait()
            pltpu.async_copy(tau_hbm.at[ihi_v], g_thi, sem).wait()
            pltpu.async_copy(dt_hbm.at[ilo_v], g_dlo, sem).wait()
            pltpu.async_copy(dt_hbm.at[ihi_v], g_dhi, sem).wait()
            zero = jnp.zeros((_L,), jnp.float32)
            mlo = ev > sv
            mhi = (ev - sv) >= 2
            corr = (jnp.where(mlo, _term(g_tlo[...], g_dlo[...]), zero)
                    + jnp.where(mhi, _term(g_thi[...], g_dhi[...]), zero))
            c_sum = jnp.sum(corr)

            pltpu.sync_copy(shared, red_v)
            tot = jnp.zeros((_L,), jnp.float32)
            for r in range(_NS):
                tot = tot + red_v[pl.ds(r * _L, _L)]
            s_sum = jnp.sum(tot)

            final = (c_sum - s_sum) * (1.0 / denom)
            out_v[...] = jnp.broadcast_to(final, (_L,))
            pltpu.sync_copy(out_v, out_hbm)

    return sc_loss


def kernel(pred_next_dt, next_dt, dt, offsets):
    tau = lax.stop_gradient(pred_next_dt)
    n = tau.shape[0]
    nseg = offsets.shape[0] - 1
    pad = _L - nseg
    one = jnp.ones((pad,), jnp.int32)
    s_arr = jnp.concatenate([offsets[:-1], one])
    e_arr = jnp.concatenate([offsets[1:], one])
    out = _make_sc_call(n, nseg, float(offsets.shape[0]))(
        tau, dt[:, 0], s_arr, e_arr)
    loss = out[0]
    return (loss, loss)


# speedup vs baseline: 1.2493x; 1.1351x over previous
import functools

import jax
import jax.numpy as jnp
from jax import lax
from jax.experimental import pallas as pl
from jax.experimental.pallas import tpu as pltpu
from jax.experimental.pallas import tpu_sc as plsc

_L = 16
_NS = 16
_EPS = 1e-5
_LN2 = 0.6931471805599453
_SQRT2 = 1.4142135623730951


def _vlog(x):
    bits = lax.bitcast_convert_type(x, jnp.int32)
    e = (bits >> 23) - 127
    m = lax.bitcast_convert_type((bits & 0x007FFFFF) | 0x3F800000, jnp.float32)
    big = m > _SQRT2
    e = jnp.where(big, e + 1, e)
    m = jnp.where(big, m * 0.5, m)
    t = (m - 1.0) / (m + 1.0)
    t2 = t * t
    p = 2.0 * t * (1.0 + t2 * (1.0 / 3.0 + t2 * (0.2 + t2 * (1.0 / 7.0))))
    return e.astype(jnp.float32) * _LN2 + p


def _term(tau, dtv):
    return -_vlog(tau) - (dtv + _EPS) / tau


def _make_sc_call(n, nseg, denom):
    chunk = n // _NS
    nvec = chunk // _L
    mesh = plsc.VectorSubcoreMesh(
        core_axis_name="c", subcore_axis_name="s", num_cores=2,
        num_subcores=_NS)

    @functools.partial(
        pl.kernel,
        out_type=jax.ShapeDtypeStruct((_L,), jnp.float32),
        mesh=mesh,
        compiler_params=pltpu.CompilerParams(needs_layout_passes=False),
        scratch_types=dict(
            tau_v=pltpu.VMEM((chunk,), jnp.float32),
            dt_v=pltpu.VMEM((chunk,), jnp.float32),
            s_v=pltpu.VMEM((_L,), jnp.int32),
            e_v=pltpu.VMEM((_L,), jnp.int32),
            ilo_v=pltpu.VMEM((_L,), jnp.int32),
            ihi_v=pltpu.VMEM((_L,), jnp.int32),
            g_tlo=pltpu.VMEM((_L,), jnp.float32),
            g_thi=pltpu.VMEM((_L,), jnp.float32),
            g_dlo=pltpu.VMEM((_L,), jnp.float32),
            g_dhi=pltpu.VMEM((_L,), jnp.float32),
            acc_v=pltpu.VMEM((_L,), jnp.float32),
            red_v=pltpu.VMEM((_NS * _L,), jnp.float32),
            out_v=pltpu.VMEM((_L,), jnp.float32),
            shared=pltpu.VMEM_SHARED((_NS * _L,), jnp.float32),
            sem_chunk=pltpu.SemaphoreType.DMA,
            sem_se=pltpu.SemaphoreType.DMA,
            sem_g=pltpu.SemaphoreType.DMA,
        ),
    )
    def sc_loss(tau_hbm, dt_hbm, s_hbm, e_hbm, out_hbm, *, tau_v, dt_v, s_v,
                e_v, ilo_v, ihi_v, g_tlo, g_thi, g_dlo, g_dhi, acc_v, red_v,
                out_v, shared, sem_chunk, sem_se, sem_g):
        cid = lax.axis_index("c")
        sid = lax.axis_index("s")
        is_w0 = jnp.logical_and(cid == 0, sid == 0)

        base = sid * chunk
        c_tau = pltpu.async_copy(tau_hbm.at[pl.ds(base, chunk)], tau_v,
                                 sem_chunk)
        c_dt = pltpu.async_copy(dt_hbm.at[pl.ds(base, chunk)], dt_v,
                                sem_chunk)

        @pl.when(is_w0)
        def _prefetch_boundaries():
            a = pltpu.async_copy(s_hbm, s_v, sem_se)
            b = pltpu.async_copy(e_hbm, e_v, sem_se)
            a.wait()
            b.wait()
            sv = s_v[...]
            ev = e_v[...]
            ilo_v[...] = sv
            ihi_v[...] = jnp.maximum(ev - 1, 0)
            pltpu.async_copy(tau_hbm.at[ilo_v], g_tlo, sem_g)
            pltpu.async_copy(tau_hbm.at[ihi_v], g_thi, sem_g)
            pltpu.async_copy(dt_hbm.at[ilo_v], g_dlo, sem_g)
            pltpu.async_copy(dt_hbm.at[ihi_v], g_dhi, sem_g)

        c_tau.wait()
        c_dt.wait()
        acc = jnp.zeros((_L,), jnp.float32)
        for k in range(nvec):
            tv = tau_v[pl.ds(k * _L, _L)]
            dv = dt_v[pl.ds(k * _L, _L)]
            acc = acc + _term(tv, dv)
        acc_v[...] = acc
        pltpu.sync_copy(acc_v, shared.at[pl.ds(sid * _L, _L)])
        plsc.subcore_barrier()

        @pl.when(is_w0)
        def _finalize():
            pltpu.sync_copy(shared, red_v)
            pltpu.make_async_copy(tau_hbm.at[ilo_v], g_tlo, sem_g).wait()
            pltpu.make_async_copy(tau_hbm.at[ihi_v], g_thi, sem_g).wait()
            pltpu.make_async_copy(dt_hbm.at[ilo_v], g_dlo, sem_g).wait()
            pltpu.make_async_copy(dt_hbm.at[ihi_v], g_dhi, sem_g).wait()
            sv = s_v[...]
            ev = e_v[...]
            zero = jnp.zeros((_L,), jnp.float32)
            mlo = ev > sv
            mhi = (ev - sv) >= 2
            corr = (jnp.where(mlo, _term(g_tlo[...], g_dlo[...]), zero)
                    + jnp.where(mhi, _term(g_thi[...], g_dhi[...]), zero))
            tot = corr
            for r in range(_NS):
                tot = tot - red_v[pl.ds(r * _L, _L)]
            final = jnp.sum(tot) * (1.0 / denom)
            out_v[...] = jnp.broadcast_to(final, (_L,))
            pltpu.sync_copy(out_v, out_hbm)

    return sc_loss


def kernel(pred_next_dt, next_dt, dt, offsets):
    tau = lax.stop_gradient(pred_next_dt)
    n = tau.shape[0]
    nseg = offsets.shape[0] - 1
    pad = _L - nseg
    one = jnp.ones((pad,), jnp.int32)
    s_arr = jnp.concatenate([offsets[:-1], one])
    e_arr = jnp.concatenate([offsets[1:], one])
    out = _make_sc_call(n, nseg, float(offsets.shape[0]))(
        tau, dt[:, 0], s_arr, e_arr)
    loss = out[0]
    return (loss, loss)


# speedup vs baseline: 1.3301x; 1.0647x over previous
import functools

import jax
import jax.numpy as jnp
from jax import lax
from jax.experimental import pallas as pl
from jax.experimental.pallas import tpu as pltpu
from jax.experimental.pallas import tpu_sc as plsc

_L = 16
_NS = 16
_EPS = 1e-5
_LN2 = 0.6931471805599453
_SQRT2 = 1.4142135623730951


def _vlog(x):
    bits = lax.bitcast_convert_type(x, jnp.int32)
    e = (bits >> 23) - 127
    m = lax.bitcast_convert_type((bits & 0x007FFFFF) | 0x3F800000, jnp.float32)
    big = m > _SQRT2
    e = jnp.where(big, e + 1, e)
    m = jnp.where(big, m * 0.5, m)
    t = (m - 1.0) / (m + 1.0)
    t2 = t * t
    p = 2.0 * t * (1.0 + t2 * (1.0 / 3.0 + t2 * (0.2 + t2 * (1.0 / 7.0))))
    return e.astype(jnp.float32) * _LN2 + p


def _term(tau, dtv):
    return -_vlog(tau) - (dtv + _EPS) / tau


def _make_sc_call(n, nseg, denom):
    chunk = n // _NS
    nvec = chunk // _L
    mesh = plsc.VectorSubcoreMesh(
        core_axis_name="c", subcore_axis_name="s", num_cores=1,
        num_subcores=_NS)

    @functools.partial(
        pl.kernel,
        out_type=jax.ShapeDtypeStruct((_L,), jnp.float32),
        mesh=mesh,
        compiler_params=pltpu.CompilerParams(needs_layout_passes=False),
        scratch_types=dict(
            tau_v=pltpu.VMEM((chunk,), jnp.float32),
            dt_v=pltpu.VMEM((chunk,), jnp.float32),
            s_v=pltpu.VMEM((_L,), jnp.int32),
            e_v=pltpu.VMEM((_L,), jnp.int32),
            ilo_v=pltpu.VMEM((_L,), jnp.int32),
            ihi_v=pltpu.VMEM((_L,), jnp.int32),
            g_tlo=pltpu.VMEM((_L,), jnp.float32),
            g_thi=pltpu.VMEM((_L,), jnp.float32),
            g_dlo=pltpu.VMEM((_L,), jnp.float32),
            g_dhi=pltpu.VMEM((_L,), jnp.float32),
            acc_v=pltpu.VMEM((_L,), jnp.float32),
            red_v=pltpu.VMEM((_NS * _L,), jnp.float32),
            out_v=pltpu.VMEM((_L,), jnp.float32),
            shared=pltpu.VMEM_SHARED((_NS * _L,), jnp.float32),
            sem_chunk=pltpu.SemaphoreType.DMA,
            sem_se=pltpu.SemaphoreType.DMA,
            sem_g=pltpu.SemaphoreType.DMA,
        ),
    )
    def sc_loss(tau_hbm, dt_hbm, s_hbm, e_hbm, out_hbm, *, tau_v, dt_v, s_v,
                e_v, ilo_v, ihi_v, g_tlo, g_thi, g_dlo, g_dhi, acc_v, red_v,
                out_v, shared, sem_chunk, sem_se, sem_g):
        cid = lax.axis_index("c")
        sid = lax.axis_index("s")
        is_w0 = jnp.logical_and(cid == 0, sid == 0)

        base = sid * chunk
        c_tau = pltpu.async_copy(tau_hbm.at[pl.ds(base, chunk)], tau_v,
                                 sem_chunk)
        c_dt = pltpu.async_copy(dt_hbm.at[pl.ds(base, chunk)], dt_v,
                                sem_chunk)

        @pl.when(is_w0)
        def _prefetch_boundaries():
            a = pltpu.async_copy(s_hbm, s_v, sem_se)
            b = pltpu.async_copy(e_hbm, e_v, sem_se)
            a.wait()
            b.wait()
            sv = s_v[...]
            ev = e_v[...]
            ilo_v[...] = sv
            ihi_v[...] = jnp.maximum(ev - 1, 0)
            pltpu.async_copy(tau_hbm.at[ilo_v], g_tlo, sem_g)
            pltpu.async_copy(tau_hbm.at[ihi_v], g_thi, sem_g)
            pltpu.async_copy(dt_hbm.at[ilo_v], g_dlo, sem_g)
            pltpu.async_copy(dt_hbm.at[ihi_v], g_dhi, sem_g)

        c_tau.wait()
        c_dt.wait()
        acc = jnp.zeros((_L,), jnp.float32)
        for k in range(nvec):
            tv = tau_v[pl.ds(k * _L, _L)]
            dv = dt_v[pl.ds(k * _L, _L)]
            acc = acc + _term(tv, dv)
        acc_v[...] = acc
        pltpu.sync_copy(acc_v, shared.at[pl.ds(sid * _L, _L)])
        plsc.subcore_barrier()

        @pl.when(is_w0)
        def _finalize():
            pltpu.sync_copy(shared, red_v)
            pltpu.make_async_copy(tau_hbm.at[ilo_v], g_tlo, sem_g).wait()
            pltpu.make_async_copy(tau_hbm.at[ihi_v], g_thi, sem_g).wait()
            pltpu.make_async_copy(dt_hbm.at[ilo_v], g_dlo, sem_g).wait()
            pltpu.make_async_copy(dt_hbm.at[ihi_v], g_dhi, sem_g).wait()
            sv = s_v[...]
            ev = e_v[...]
            zero = jnp.zeros((_L,), jnp.float32)
            mlo = ev > sv
            mhi = (ev - sv) >= 2
            corr = (jnp.where(mlo, _term(g_tlo[...], g_dlo[...]), zero)
                    + jnp.where(mhi, _term(g_thi[...], g_dhi[...]), zero))
            tot = corr
            for r in range(_NS):
                tot = tot - red_v[pl.ds(r * _L, _L)]
            final = jnp.sum(tot) * (1.0 / denom)
            out_v[...] = jnp.broadcast_to(final, (_L,))
            pltpu.sync_copy(out_v, out_hbm)

    return sc_loss


def kernel(pred_next_dt, next_dt, dt, offsets):
    tau = lax.stop_gradient(pred_next_dt)
    n = tau.shape[0]
    nseg = offsets.shape[0] - 1
    pad = _L - nseg
    one = jnp.ones((pad,), jnp.int32)
    s_arr = jnp.concatenate([offsets[:-1], one])
    e_arr = jnp.concatenate([offsets[1:], one])
    out = _make_sc_call(n, nseg, float(offsets.shape[0]))(
        tau, dt[:, 0], s_arr, e_arr)
    loss = out[0]
    return (loss, loss)
